# baseline (speedup 1.0000x reference)
import functools

import jax
import jax.numpy as jnp
from jax import lax
from jax.experimental import pallas as pl
from jax.experimental.pallas import tpu as pltpu
from jax.experimental.pallas import tpu_sc as plsc

N, H, W, C = 4, 384, 384, 96
Q = N * H * W
CW = 128
HB = 128
NW = 32
QW = Q // NW
GM = 128
G = 64
NB2 = QW // G
NBT = Q // GM
LANES = 16
R = C // LANES


def _prep_body(flow_ref, i00_ref, i01_ref, i10_ref, i11_ref, ax_ref, ay_ref):
    n = pl.program_id(0)
    hb = pl.program_id(1)
    fy = flow_ref[0, 0]
    fx = flow_ref[1, 0]
    yy = (hb * HB + lax.broadcasted_iota(jnp.int32, (HB, W), 0)).astype(jnp.float32)
    xx = lax.broadcasted_iota(jnp.int32, (HB, W), 1).astype(jnp.float32)
    qy = yy - fy
    qx = xx - fx
    fly = jnp.clip(jnp.floor(qy), 0.0, float(H - 2))
    flx = jnp.clip(jnp.floor(qx), 0.0, float(W - 2))
    ay_ref[0] = jnp.clip(qy - fly, 0.0, 1.0)
    ax_ref[0] = jnp.clip(qx - flx, 0.0, 1.0)
    iy = fly.astype(jnp.int32)
    ix = flx.astype(jnp.int32)
    lin = (n * H + iy) * W + ix
    i00_ref[0] = lin
    i01_ref[0] = lin + 1
    i10_ref[0] = lin + W
    i11_ref[0] = lin + W + 1


_prep = pl.pallas_call(
    _prep_body,
    grid=(N, H // HB),
    in_specs=[pl.BlockSpec((2, 1, HB, W), lambda n, h: (0, n, h, 0))],
    out_specs=[pl.BlockSpec((1, HB, W), lambda n, h: (n, h, 0))] * 6,
    out_shape=[jax.ShapeDtypeStruct((N, H, W), jnp.int32)] * 4
    + [jax.ShapeDtypeStruct((N, H, W), jnp.float32)] * 2,
)


def _blend_block(mv, qoff, nb0, nb1, nb2, nb3, out_v):
    k4 = jnp.full((LANES,), 4, jnp.int32)
    k5 = jnp.full((LANES,), 5, jnp.int32)

    @plsc.parallel_loop(0, G, unroll=4)
    def _query(q):
        qv = jnp.full((LANES,), qoff, jnp.int32) + q
        axq = plsc.bitcast(plsc.load_gather(mv, [k4, qv]), jnp.float32)
        ayq = plsc.bitcast(plsc.load_gather(mv, [k5, qv]), jnp.float32)
        for r in range(R):
            cs = pl.ds(r * LANES, LANES)
            tl = nb0[q, cs]
            tr = nb1[q, cs]
            bl = nb2[q, cs]
            br = nb3[q, cs]
            top = tl + axq * (tr - tl)
            bot = bl + axq * (br - bl)
            out_v[q, cs] = top + ayq * (bot - top)


def _post_body(in_ref, out_ref):
    for i in range(8):
        out_ref[0, i] = in_ref[pl.ds(i * W, W), :C]


_post = pl.pallas_call(
    _post_body,
    grid=(N, H // 8),
    in_specs=[pl.BlockSpec((8 * W, CW), lambda n, h: ((n * (H // 8) + h), 0))],
    out_specs=pl.BlockSpec((1, 8, W, C), lambda n, h: (n, h, 0, 0)),
    out_shape=jax.ShapeDtypeStruct((N, H, W, C), jnp.float32),
)


def _warp_sc_body(img_h, meta_h, out_h,
                  m0, m1, m2, a0, a1, a2, a3, b0, b1, b2, b3, o0, o1,
                  ms0, ms1, ms2, gs0, gs1, os0, os1):
    wid = lax.axis_index("s") * 2 + lax.axis_index("c")
    mblk0 = wid * (NB2 // 2)
    row0 = wid * QW
    metas = (m0, m1, m2)
    msems = (ms0, ms1, ms2)
    nbufs = ((a0, a1, a2, a3), (b0, b1, b2, b3))
    gsems = (gs0, gs1)
    obufs = (o0, o1)
    osems = (os0, os1)

    def meta_copy(m, slot):
        return pltpu.make_async_copy(meta_h.at[mblk0 + m], metas[slot], msems[slot])

    def gather_copies(par, mslot, g_slot):
        idx = metas[mslot].at[:, pl.ds(par * G, G)]
        return [
            pltpu.make_async_copy(
                img_h.at[idx.at[k]], nbufs[g_slot][k], gsems[g_slot]
            )
            for k in range(4)
        ]

    def out_copy(b, slot):
        return pltpu.make_async_copy(
            obufs[slot], out_h.at[pl.ds(row0 + b * G, G)], osems[slot]
        )

    meta_copy(0, 0).start()
    meta_copy(0, 0).wait()
    meta_copy(1, 1).start()
    for cp in gather_copies(0, 0, 0):
        cp.start()

    @pl.loop(0, NB2, step=12)
    def _outer(ob):
        for ph in range(12):
            b = ob + ph
            sb = ph % 2
            sn = 1 - sb
            mcur = (ph // 2) % 3
            mnext = ((ph + 1) // 2) % 3

            @pl.when(b + 1 < NB2)
            def _issue_next():
                if (ph + 1) % 2 == 0:
                    meta_copy((b + 1) // 2, mnext).wait()
                for cp in gather_copies((ph + 1) % 2, mnext, sn):
                    cp.start()

            if ph % 2 == 0:
                @pl.when(b // 2 + 2 < NB2 // 2)
                def _issue_meta():
                    meta_copy(b // 2 + 2, (ph // 2 + 2) % 3).start()

            for cp in gather_copies(ph % 2, mcur, sb):
                cp.wait()

            @pl.when(b >= 2)
            def _wait_out():
                out_copy(b - 2, sb).wait()

            nb0, nb1, nb2, nb3 = nbufs[sb]
            _blend_block(metas[mcur], (ph % 2) * G, nb0, nb1, nb2, nb3,
                         obufs[sb])
            out_copy(b, sb).start()

    out_copy(NB2 - 2, 0).wait()
    out_copy(NB2 - 1, 1).wait()


@functools.cache
def _get_warp_sc():
    return pl.kernel(
        _warp_sc_body,
        out_type=jax.ShapeDtypeStruct((Q, CW), jnp.float32),
        mesh=plsc.VectorSubcoreMesh(core_axis_name="c", subcore_axis_name="s"),
        scratch_types=[pltpu.VMEM((8, GM), jnp.int32)] * 3
        + [pltpu.VMEM((G, CW), jnp.float32)] * 8
        + [pltpu.VMEM((G, CW), jnp.float32)] * 2
        + [pltpu.SemaphoreType.DMA] * 7,
        compiler_params=pltpu.CompilerParams(
            needs_layout_passes=False, use_tc_tiling_on_sc=False
        ),
    )


def kernel(image, flow):
    flow_t = jnp.moveaxis(flow, 3, 0)
    i00, i01, i10, i11, ax, ay = _prep(flow_t)
    mz = jnp.zeros((NBT, GM), jnp.int32)
    meta = jnp.stack(
        [
            i00.reshape(NBT, GM),
            i01.reshape(NBT, GM),
            i10.reshape(NBT, GM),
            i11.reshape(NBT, GM),
            lax.bitcast_convert_type(ax, jnp.int32).reshape(NBT, GM),
            lax.bitcast_convert_type(ay, jnp.int32).reshape(NBT, GM),
            mz,
            mz,
        ],
        axis=1,
    )
    img128 = jnp.pad(image, ((0, 0), (0, 0), (0, 0), (0, CW - C))).reshape(Q, CW)
    out = _get_warp_sc()(img128, meta)
    return _post(out)


# speedup vs baseline: 1.2246x; 1.2246x over previous
import functools

import jax
import jax.numpy as jnp
from jax import lax
from jax.experimental import pallas as pl
from jax.experimental.pallas import tpu as pltpu
from jax.experimental.pallas import tpu_sc as plsc

N, H, W, C = 4, 384, 384, 96
Q = N * H * W
CW = 128
HB = 128
NW = 32
QW = Q // NW
GM = 128
G = 64
NB2 = QW // G
NBT = Q // GM
LANES = 16
R = C // LANES


def _prep_body(flow_ref, i00_ref, i01_ref, i10_ref, i11_ref, ax_ref, ay_ref):
    n = pl.program_id(0)
    hb = pl.program_id(1)
    fy = flow_ref[0, 0]
    fx = flow_ref[1, 0]
    yy = (hb * HB + lax.broadcasted_iota(jnp.int32, (HB, W), 0)).astype(jnp.float32)
    xx = lax.broadcasted_iota(jnp.int32, (HB, W), 1).astype(jnp.float32)
    qy = yy - fy
    qx = xx - fx
    fly = jnp.clip(jnp.floor(qy), 0.0, float(H - 2))
    flx = jnp.clip(jnp.floor(qx), 0.0, float(W - 2))
    ay_ref[0] = jnp.clip(qy - fly, 0.0, 1.0)
    ax_ref[0] = jnp.clip(qx - flx, 0.0, 1.0)
    iy = fly.astype(jnp.int32)
    ix = flx.astype(jnp.int32)
    lin = (n * H + iy) * W + ix
    i00_ref[0] = lin
    i01_ref[0] = lin + 1
    i10_ref[0] = lin + W
    i11_ref[0] = lin + W + 1


_prep = pl.pallas_call(
    _prep_body,
    grid=(N, H // HB),
    in_specs=[pl.BlockSpec((2, 1, HB, W), lambda n, h: (0, n, h, 0))],
    out_specs=[pl.BlockSpec((1, HB, W), lambda n, h: (n, h, 0))] * 6,
    out_shape=[jax.ShapeDtypeStruct((N, H, W), jnp.int32)] * 4
    + [jax.ShapeDtypeStruct((N, H, W), jnp.float32)] * 2,
)


def _blend_block(mv, qoff, nb0, nb1, nb2, nb3, out_v):
    k4 = jnp.full((LANES,), 4, jnp.int32)
    k5 = jnp.full((LANES,), 5, jnp.int32)

    @plsc.parallel_loop(0, G, unroll=4)
    def _query(q):
        qv = jnp.full((LANES,), qoff, jnp.int32) + q
        axq = plsc.bitcast(plsc.load_gather(mv, [k4, qv]), jnp.float32)
        ayq = plsc.bitcast(plsc.load_gather(mv, [k5, qv]), jnp.float32)
        for r in range(R):
            cs = pl.ds(r * LANES, LANES)
            tl = nb0[q, cs]
            tr = nb1[q, cs]
            bl = nb2[q, cs]
            br = nb3[q, cs]
            top = tl + axq * (tr - tl)
            bot = bl + axq * (br - bl)
            out_v[q, cs] = top + ayq * (bot - top)


def _post_body(in_ref, out_ref):
    for i in range(8):
        out_ref[0, i] = in_ref[pl.ds(i * W, W), :C]


_post = pl.pallas_call(
    _post_body,
    grid=(N, H // 8),
    in_specs=[pl.BlockSpec((8 * W, CW), lambda n, h: ((n * (H // 8) + h), 0))],
    out_specs=pl.BlockSpec((1, 8, W, C), lambda n, h: (n, h, 0, 0)),
    out_shape=jax.ShapeDtypeStruct((N, H, W, C), jnp.float32),
)


def _warp_sc_body(img_h, meta_h, out_h,
                  m0, m1, m2, a0, a1, a2, a3, b0, b1, b2, b3, o0, o1,
                  ms0, ms1, ms2, gs0, gs1, os0, os1):
    wid = lax.axis_index("s") * 2 + lax.axis_index("c")
    mblk0 = wid * (NB2 // 2)
    row0 = wid * QW
    metas = (m0, m1, m2)
    msems = (ms0, ms1, ms2)
    nbufs = ((a0, a1, a2, a3), (b0, b1, b2, b3))
    gsems = (gs0, gs1)
    obufs = (o0, o1)
    osems = (os0, os1)

    def meta_copy(m, slot):
        return pltpu.make_async_copy(meta_h.at[mblk0 + m], metas[slot], msems[slot])

    def gather_copies(par, mslot, g_slot):
        idx = metas[mslot].at[:, pl.ds(par * G, G)]
        return [
            pltpu.make_async_copy(
                img_h.at[idx.at[k]], nbufs[g_slot][k], gsems[g_slot]
            )
            for k in range(4)
        ]

    def out_copy(b, slot):
        return pltpu.make_async_copy(
            obufs[slot], out_h.at[pl.ds(row0 + b * G, G)], osems[slot]
        )

    meta_copy(0, 0).start()
    meta_copy(0, 0).wait()
    meta_copy(1, 1).start()
    for cp in gather_copies(0, 0, 0):
        cp.start()

    @pl.loop(0, NB2, step=12)
    def _outer(ob):
        for ph in range(12):
            b = ob + ph
            sb = ph % 2
            sn = 1 - sb
            mcur = (ph // 2) % 3
            mnext = ((ph + 1) // 2) % 3

            @pl.when(b + 1 < NB2)
            def _issue_next():
                if (ph + 1) % 2 == 0:
                    meta_copy((b + 1) // 2, mnext).wait()
                for cp in gather_copies((ph + 1) % 2, mnext, sn):
                    cp.start()

            if ph % 2 == 0:
                @pl.when(b // 2 + 2 < NB2 // 2)
                def _issue_meta():
                    meta_copy(b // 2 + 2, (ph // 2 + 2) % 3).start()

            for cp in gather_copies(ph % 2, mcur, sb):
                cp.wait()

            @pl.when(b >= 2)
            def _wait_out():
                out_copy(b - 2, sb).wait()

            nb0, nb1, nb2, nb3 = nbufs[sb]
            _blend_block(metas[mcur], (ph % 2) * G, nb0, nb1, nb2, nb3,
                         obufs[sb])
            out_copy(b, sb).start()

    out_copy(NB2 - 2, 0).wait()
    out_copy(NB2 - 1, 1).wait()


@functools.cache
def _get_warp_sc():
    return pl.kernel(
        _warp_sc_body,
        out_type=jax.ShapeDtypeStruct((Q, CW), jnp.float32),
        mesh=plsc.VectorSubcoreMesh(core_axis_name="c", subcore_axis_name="s"),
        scratch_types=[pltpu.VMEM((8, GM), jnp.int32)] * 3
        + [pltpu.VMEM((G, CW), jnp.float32)] * 8
        + [pltpu.VMEM((G, CW), jnp.float32)] * 2
        + [pltpu.SemaphoreType.DMA] * 7,
        compiler_params=pltpu.CompilerParams(
            needs_layout_passes=False, use_tc_tiling_on_sc=False
        ),
    )


def kernel(image, flow):
    flow_t = jnp.moveaxis(flow, 3, 0)
    i00, i01, i10, i11, ax, ay = _prep(flow_t)
    mz = jnp.zeros((NBT, GM), jnp.int32)
    meta = jnp.stack(
        [
            i00.reshape(NBT, GM),
            i01.reshape(NBT, GM),
            i10.reshape(NBT, GM),
            i11.reshape(NBT, GM),
            lax.bitcast_convert_type(ax, jnp.int32).reshape(NBT, GM),
            lax.bitcast_convert_type(ay, jnp.int32).reshape(NBT, GM),
            mz,
            mz,
        ],
        axis=1,
    )
    img128 = jnp.pad(image, ((0, 0), (0, 0), (0, 0), (0, CW - C))).reshape(Q, CW)
    out = _get_warp_sc()(img128, meta)
    return out.reshape(N, H, W, CW)[..., :C]
